# baseline (speedup 1.0000x reference)
import functools

import jax
import jax.numpy as jnp
from jax import lax
from jax.experimental import pallas as pl
from jax.experimental.pallas import tpu as pltpu
from jax.experimental.pallas import tpu_sc as plsc

N = 4096
KK = 2048
D = 512


_IC = 512
_JC = 256


def _topk_body(srow_ref, scol_ref, idx_ref, val_ref, ranks_ref):
    def rank_chunk(ci, carry):
        sch = srow_ref[0:1, pl.ds(ci * _IC, _IC)]
        i_ids = jax.lax.broadcasted_iota(jnp.int32, (_JC, _IC), 1) + ci * _IC

        def jbody(cj, acc):
            sj = scol_ref[pl.ds(cj * _JC, _JC), 0:1]
            j_ids = jax.lax.broadcasted_iota(jnp.int32, (_JC, _IC), 0) + cj * _JC
            gt = (sj > sch).astype(jnp.float32)
            eq = jnp.logical_and(sj == sch, j_ids < i_ids).astype(jnp.float32)
            return acc + jnp.sum(gt + eq, axis=0, keepdims=True)

        rank = jax.lax.fori_loop(0, N // _JC, jbody,
                                 jnp.zeros((1, _IC), jnp.float32))
        ranks_ref[0:1, pl.ds(ci * _IC, _IC)] = rank
        return carry

    jax.lax.fori_loop(0, N // _IC, rank_chunk, 0)

    def inv_chunk(cr, carry):
        r_col = (jax.lax.broadcasted_iota(jnp.int32, (_JC, _IC), 0)
                 + cr * _JC).astype(jnp.float32)

        def jbody(cj, accs):
            acc_i, acc_v = accs
            rk = ranks_ref[0:1, pl.ds(cj * _IC, _IC)]
            sc = srow_ref[0:1, pl.ds(cj * _IC, _IC)]
            j_ids = (jax.lax.broadcasted_iota(jnp.int32, (_JC, _IC), 1)
                     + cj * _IC).astype(jnp.float32)
            m = rk == r_col
            acc_i = acc_i + jnp.sum(jnp.where(m, j_ids, 0.0), axis=1,
                                    keepdims=True)
            acc_v = acc_v + jnp.sum(jnp.where(m, sc, 0.0), axis=1,
                                    keepdims=True)
            return acc_i, acc_v

        z = jnp.zeros((_JC, 1), jnp.float32)
        acc_i, acc_v = jax.lax.fori_loop(0, N // _IC, jbody, (z, z))
        idx_ref[pl.ds(cr * _JC, _JC), 0:1] = acc_i.astype(jnp.int32)
        val_ref[pl.ds(cr * _JC, _JC), 0:1] = acc_v
        return carry

    jax.lax.fori_loop(0, KK // _JC, inv_chunk, 0)


def _topk_call(srow, scol):
    return pl.pallas_call(
        _topk_body,
        out_shape=[jax.ShapeDtypeStruct((KK, 1), jnp.int32),
                   jax.ShapeDtypeStruct((KK, 1), jnp.float32)],
        scratch_shapes=[pltpu.VMEM((1, N), jnp.float32)],
    )(srow, scol)


_TB = 512


def _prep_body(g_ref, gb_ref, gt_ref):
    blk = g_ref[...]
    gb_ref[...] = (blk != 0).astype(jnp.bfloat16)
    gt_ref[...] = blk.T


def _prep_call(g):
    nb = N // _TB
    return pl.pallas_call(
        _prep_body,
        grid=(nb, nb),
        in_specs=[pl.BlockSpec((_TB, _TB), lambda i, j: (i, j))],
        out_specs=[pl.BlockSpec((_TB, _TB), lambda i, j: (i, j)),
                   pl.BlockSpec((_TB, _TB), lambda i, j: (j, i))],
        out_shape=[jax.ShapeDtypeStruct((N, N), jnp.bfloat16),
                   jax.ShapeDtypeStruct((N, N), jnp.float32)],
    )(g)


_NC, _NS = 2, 16
_NW = _NC * _NS
_RPW = KK // _NW
_CH = 8


def _gather_body(g_hbm, gt_hbm, h_hbm, idx_hbm, idx2_hbm,
                 o_gs, o_gts, o_hs,
                 idx_v, idx2_v, buf_g, buf_gt, buf_h,
                 sem_g, sem_gt, sem_h):
    wid = lax.axis_index("s") * _NC + lax.axis_index("c")
    base = wid * _RPW

    pltpu.sync_copy(idx_hbm.at[pl.ds(base, _RPW)], idx_v)
    nch = _RPW // _CH
    pltpu.sync_copy(idx2_hbm.at[pl.ds(wid * nch, nch)], idx2_v)

    pltpu.async_copy(h_hbm.at[idx_v], buf_h, sem_h).wait()
    pltpu.sync_copy(buf_h, o_hs.at[pl.ds(base, _RPW)])

    def chunk(c, carry):
        cp_g = pltpu.async_copy(g_hbm.at[idx2_v.at[c]], buf_g, sem_g)
        cp_t = pltpu.async_copy(gt_hbm.at[idx2_v.at[c]], buf_gt, sem_gt)
        cp_g.wait()
        pltpu.sync_copy(buf_g, o_gs.at[pl.ds(base + c * _CH, _CH)])
        cp_t.wait()
        pltpu.sync_copy(buf_gt, o_gts.at[pl.ds(base + c * _CH, _CH)])
        return carry

    jax.lax.fori_loop(0, nch, chunk, 0)


def _gather_call(g, gT, h, idx, idx2):
    mesh = plsc.VectorSubcoreMesh(core_axis_name="c", subcore_axis_name="s")
    f = functools.partial(
        pl.kernel,
        out_type=[jax.ShapeDtypeStruct((KK, N), jnp.float32),
                  jax.ShapeDtypeStruct((KK, N), jnp.float32),
                  jax.ShapeDtypeStruct((KK, D), jnp.float32)],
        mesh=mesh,
        scratch_types=[pltpu.VMEM((_RPW,), jnp.int32),
                       pltpu.VMEM((_RPW // _CH, _CH), jnp.int32),
                       pltpu.VMEM((_CH, N), jnp.float32),
                       pltpu.VMEM((_CH, N), jnp.float32),
                       pltpu.VMEM((_RPW, D), jnp.float32),
                       pltpu.SemaphoreType.DMA,
                       pltpu.SemaphoreType.DMA,
                       pltpu.SemaphoreType.DMA],
    )(_gather_body)
    return f(g, gT, h, idx, idx2)


_M1_BM, _M1_BN, _M1_BK = 512, 1024, 512


def _mm1_body(a_ref, b_ref, o_ref, acc_ref):
    @pl.when(pl.program_id(2) == 0)
    def _():
        acc_ref[...] = jnp.zeros_like(acc_ref)

    a = (a_ref[...] != 0).astype(jnp.bfloat16)
    acc_ref[...] += jnp.dot(a, b_ref[...],
                            preferred_element_type=jnp.float32)

    @pl.when(pl.program_id(2) == N // _M1_BK - 1)
    def _():
        o_ref[...] = (acc_ref[...] > 0.5).astype(jnp.bfloat16)


def _mm1_call(gs, gb16):
    grid = (KK // _M1_BM, N // _M1_BN, N // _M1_BK)
    return pl.pallas_call(
        _mm1_body,
        grid=grid,
        in_specs=[pl.BlockSpec((_M1_BM, _M1_BK), lambda i, j, k: (i, k)),
                  pl.BlockSpec((_M1_BK, _M1_BN), lambda i, j, k: (k, j))],
        out_specs=pl.BlockSpec((_M1_BM, _M1_BN), lambda i, j, k: (i, j)),
        out_shape=jax.ShapeDtypeStruct((KK, N), jnp.bfloat16),
        scratch_shapes=[pltpu.VMEM((_M1_BM, _M1_BN), jnp.float32)],
        compiler_params=pltpu.CompilerParams(
            dimension_semantics=("parallel", "parallel", "arbitrary")),
    )(gs, gb16)


_M2_BM, _M2_BK = 256, 256


def _mm2_body(x_ref, t_ref, o_ref, acc_ref):
    @pl.when(pl.program_id(1) == 0)
    def _():
        acc_ref[...] = jnp.zeros_like(acc_ref)

    t = (t_ref[...] != 0).astype(jnp.bfloat16)
    acc_ref[...] += jax.lax.dot_general(
        x_ref[...], t, (((1,), (1,)), ((), ())),
        preferred_element_type=jnp.float32)

    @pl.when(pl.program_id(1) == N // _M2_BK - 1)
    def _():
        un = (acc_ref[...] > 0.5).astype(jnp.float32)
        s = jnp.sum(un, axis=1, keepdims=True)
        o_ref[...] = un / (s + 1e-08)


def _mm2_call(xb, gts):
    grid = (KK // _M2_BM, N // _M2_BK)
    return pl.pallas_call(
        _mm2_body,
        grid=grid,
        in_specs=[pl.BlockSpec((_M2_BM, _M2_BK), lambda i, k: (i, k)),
                  pl.BlockSpec((KK, _M2_BK), lambda i, k: (0, k))],
        out_specs=pl.BlockSpec((_M2_BM, KK), lambda i, k: (i, 0)),
        out_shape=jax.ShapeDtypeStruct((KK, KK), jnp.float32),
        scratch_shapes=[pltpu.VMEM((_M2_BM, KK), jnp.float32)],
        compiler_params=pltpu.CompilerParams(
            dimension_semantics=("parallel", "arbitrary")),
    )(xb, gts)


def _scale_body(h_ref, v_ref, o_ref):
    o_ref[...] = h_ref[...] * v_ref[...]


def _scale_call(hs, val2):
    bm = 256
    return pl.pallas_call(
        _scale_body,
        grid=(KK // bm,),
        in_specs=[pl.BlockSpec((bm, D), lambda i: (i, 0)),
                  pl.BlockSpec((bm, 1), lambda i: (i, 0))],
        out_specs=pl.BlockSpec((bm, D), lambda i: (i, 0)),
        out_shape=jax.ShapeDtypeStruct((KK, D), jnp.float32),
    )(hs, val2)


def kernel(g, h, C, Wf, bf, Ws, bs, Wo, bo):
    feature_weights = h @ Wf + bf
    structure_weights = C @ Ws + bs
    weights = (jnp.concatenate([feature_weights, structure_weights], axis=1)
               @ Wo + bo).squeeze()
    scores = jax.nn.sigmoid(weights)

    srow = scores.reshape(1, N)
    scol = scores.reshape(N, 1)
    idx2, val2 = _topk_call(srow, scol)
    idx = idx2.reshape(KK)

    gb16, gT = _prep_call(g)
    gs, gts, hs = _gather_call(g, gT, h, idx, idx.reshape(-1, _CH))
    xb = _mm1_call(gs, gb16)
    g_new = _mm2_call(xb, gts)
    new_h = _scale_call(hs, val2)
    return (g_new, new_h, idx)
